# Initial kernel scaffold; baseline (speedup 1.0000x reference)
import jax
import jax.numpy as jnp
from jax.experimental import pallas as pl


def kernel(x_in, Wf, bf, edge_alpha, edge_beta):
    raise NotImplementedError("write your pallas kernel here")


# speedup vs baseline: 39.6311x; 39.6311x over previous
import numpy as np
import jax
import jax.numpy as jnp
from jax.experimental import pallas as pl
from jax.experimental.pallas import tpu as pltpu

DIM = 192
WS = 7
KNN = 15
PW = 32
NPTS = PW * PW
C8 = DIM // 8
AUG = 32


def _grid_const():
    gi, gj = np.meshgrid(np.arange(PW, dtype=np.float32),
                         np.arange(PW, dtype=np.float32), indexing="ij")
    grid = np.stack([gi, gj], axis=-1).reshape(NPTS, 2)
    mean = grid.mean(0)
    std = grid.std(0, ddof=1)
    return ((grid - mean) / (std + 1e-5)).astype(np.float32)


_GRID = _grid_const()


def _patch_body(ab_ref, x_ref, wft_ref, add_ref, out_ref):
    x = x_ref[0]
    aug = jnp.dot(x, wft_ref[...], preferred_element_type=jnp.float32) + add_ref[...]
    nrm = jnp.maximum(jnp.sqrt(jnp.sum(aug * aug, axis=1, keepdims=True)), 1e-8)
    xn = aug / nrm
    s = jax.lax.dot_general(xn, xn, (((1,), (1,)), ((), ())),
                            preferred_element_type=jnp.float32)
    alpha = ab_ref[0]
    beta = ab_ref[1]
    work = s
    for _ in range(KNN - 1):
        m = jnp.max(work, axis=1, keepdims=True)
        work = jnp.where(work >= m, -3.0e38, work)
    thr = jnp.max(work, axis=1, keepdims=True)
    keep = s >= thr
    w = 1.0 / (1.0 + jnp.exp(-(beta + alpha * s)))
    e = jnp.where(keep, jnp.exp(w), 0.0)
    p = e / jnp.sum(e, axis=1, keepdims=True)
    out_ref[0] = jnp.dot(p, x, preferred_element_type=jnp.float32)


def kernel(x_in, Wf, bf, edge_alpha, edge_beta):
    B, C, H, Wd = x_in.shape
    xp = (x_in.reshape(B, DIM, WS, PW, WS, PW)
          .transpose(0, 2, 4, 3, 5, 1)
          .reshape(WS * WS, NPTS, DIM))
    wft = jnp.zeros((DIM, AUG), jnp.float32).at[:, :C8].set(Wf.T)
    add = (jnp.zeros((NPTS, AUG), jnp.float32)
           .at[:, :C8].set(bf[None, :])
           .at[:, C8:C8 + 2].set(jnp.asarray(_GRID)))
    ab = jnp.stack([edge_alpha[0], edge_beta[0]])
    out = pl.pallas_call(
        _patch_body,
        grid=(WS * WS,),
        in_specs=[
            pl.BlockSpec(memory_space=pltpu.SMEM),
            pl.BlockSpec((1, NPTS, DIM), lambda i: (i, 0, 0)),
            pl.BlockSpec((DIM, AUG), lambda i: (0, 0)),
            pl.BlockSpec((NPTS, AUG), lambda i: (0, 0)),
        ],
        out_specs=pl.BlockSpec((1, NPTS, DIM), lambda i: (i, 0, 0)),
        out_shape=jax.ShapeDtypeStruct((WS * WS, NPTS, DIM), jnp.float32),
    )(ab, xp, wft, add)
    out = (out.reshape(B, WS, WS, PW, PW, DIM)
           .transpose(0, 5, 1, 3, 2, 4)
           .reshape(B, DIM, H, Wd))
    return out


# speedup vs baseline: 54.9025x; 1.3853x over previous
import numpy as np
import jax
import jax.numpy as jnp
from jax.experimental import pallas as pl
from jax.experimental.pallas import tpu as pltpu

DIM = 192
WS = 7
KNN = 15
PW = 32
NPTS = PW * PW
C8 = DIM // 8
Wd_ = WS * PW


def _grid_const():
    gi, gj = np.meshgrid(np.arange(PW, dtype=np.float32),
                         np.arange(PW, dtype=np.float32), indexing="ij")
    grid = np.stack([gi, gj], axis=-1).reshape(NPTS, 2)
    mean = grid.mean(0)
    std = grid.std(0, ddof=1)
    return ((grid - mean) / (std + 1e-5)).astype(np.float32)


_GRID2 = _grid_const()


def _body(ab_ref, x_ref, wf_ref, bf_ref, grid_ref, out_ref, xt_scr, ot_scr):
    hg = pl.program_id(1)
    alpha = ab_ref[0]
    beta = ab_ref[1]

    @pl.when(hg == 0)
    def _():
        for j in range(WS):
            t = x_ref[:, 0, :, j * PW:(j + 1) * PW]
            xt_scr[:, j * NPTS:(j + 1) * NPTS] = t.reshape(DIM, NPTS)

    off = pl.multiple_of(hg * NPTS, NPTS)
    x = xt_scr[:, pl.ds(off, NPTS)]
    f = jax.lax.dot_general(x, wf_ref[...], (((0,), (1,)), ((), ())),
                            preferred_element_type=jnp.float32)
    f = f + bf_ref[...]
    aug = jnp.concatenate([f, grid_ref[...]], axis=1)
    nrm = jnp.maximum(jnp.sqrt(jnp.sum(aug * aug, axis=1, keepdims=True)), 1e-8)
    xn = aug / nrm
    s = jax.lax.dot_general(xn, xn, (((1,), (1,)), ((), ())),
                            preferred_element_type=jnp.float32)

    def _edge_e(v):
        return jnp.exp(1.0 / (1.0 + jnp.exp(-(beta + alpha * v))))

    m = jnp.max(s, axis=1, keepdims=True)
    den = _edge_e(m)
    for _ in range(KNN - 1):
        m = jnp.max(jnp.where(s < m, s, -3.0e38), axis=1, keepdims=True)
        den = den + _edge_e(m)
    p = jnp.where(s >= m, _edge_e(s), 0.0) / den
    o = jax.lax.dot_general(x, p, (((1,), (1,)), ((), ())),
                            preferred_element_type=jnp.float32)
    ot_scr[:, pl.ds(off, NPTS)] = o

    @pl.when(hg == WS - 1)
    def _():
        for j in range(WS):
            oj = ot_scr[:, j * NPTS:(j + 1) * NPTS].reshape(DIM, PW, PW)
            out_ref[:, 0, :, j * PW:(j + 1) * PW] = oj


def kernel(x_in, Wf, bf, edge_alpha, edge_beta):
    B, C, H, Wd = x_in.shape
    ab = jnp.stack([edge_alpha[0], edge_beta[0]])
    bf2 = bf.reshape(1, C8)
    grid2 = jnp.asarray(_GRID2)
    xs = x_in.reshape(DIM, WS, PW, Wd)
    out = pl.pallas_call(
        _body,
        grid=(WS, WS),
        in_specs=[
            pl.BlockSpec(memory_space=pltpu.SMEM),
            pl.BlockSpec((DIM, 1, PW, Wd), lambda i, j: (0, i, 0, 0)),
            pl.BlockSpec((C8, DIM), lambda i, j: (0, 0)),
            pl.BlockSpec((1, C8), lambda i, j: (0, 0)),
            pl.BlockSpec((NPTS, 2), lambda i, j: (0, 0)),
        ],
        out_specs=pl.BlockSpec((DIM, 1, PW, Wd), lambda i, j: (0, i, 0, 0)),
        out_shape=jax.ShapeDtypeStruct((DIM, WS, PW, Wd), jnp.float32),
        scratch_shapes=[
            pltpu.VMEM((DIM, WS * NPTS), jnp.float32),
            pltpu.VMEM((DIM, WS * NPTS), jnp.float32),
        ],
    )(ab, xs, Wf, bf2, grid2)
    return out.reshape(B, C, H, Wd)
